# baseline (speedup 1.0000x reference)
import functools

import jax
import jax.numpy as jnp
from jax import lax
from jax.experimental import pallas as pl
from jax.experimental.pallas import tpu as pltpu
from jax.experimental.pallas import tpu_sc as plsc

N = 10000
E = 320000
D_IN = 128
D_EDGE = 16
H = 10
L = 3

HP = 16
NW = 32
BE = 128
NBLK = 80
EPAD = NW * NBLK * BE
NPAD = 10240
RPS = NPAD // 16
ZB = 64
NDEP = 4

ER8 = E // 8
EP8 = EPAD // 8
NR8 = N // 8
NP8 = NPAD // 8

_f32 = jnp.float32


_mesh = plsc.VectorSubcoreMesh(core_axis_name="c", subcore_axis_name="s")
_sc_params = pltpu.CompilerParams(use_tc_tiling_on_sc=False)


def _zero_acc(zbuf, acc, s):
    def zrow(i, _):
        zbuf[i] = jnp.zeros((HP,), _f32)
        return 0

    lax.fori_loop(0, ZB, zrow, 0)

    def zcp(k, _):
        pltpu.sync_copy(zbuf, acc.at[pl.ds(s * RPS + k * ZB, ZB)])
        return 0

    lax.fori_loop(0, RPS // ZB, zcp, 0)


def _flush_acc(acc, out_hbm, c, s):
    pltpu.sync_copy(
        acc.at[pl.ds(s * RPS, RPS)],
        out_hbm.at[c, pl.ds(s * RPS, RPS)],
    )


@functools.partial(
    pl.kernel,
    out_type=jax.ShapeDtypeStruct((2, NPAD, HP), _f32),
    mesh=_mesh,
    compiler_params=_sc_params,
    scratch_types=[
        pltpu.VMEM((NBLK, BE), jnp.int32),
        pltpu.VMEM((NBLK, BE), jnp.int32),
        pltpu.VMEM((NDEP, BE, HP), _f32),
        pltpu.VMEM((ZB, HP), _f32),
        pltpu.VMEM_SHARED((NPAD, HP), _f32),
    ] + [pltpu.SemaphoreType.DMA] * NDEP,
)
def _seg_gather(r_hbm, sd_hbm, out_hbm, sidx, didx, rows, zbuf, acc,
                s0, s1, s2, s3):
    c = lax.axis_index("c")
    s = lax.axis_index("s")
    wid = s * 2 + c
    sems = (s0, s1, s2, s3)
    _zero_acc(zbuf, acc, s)
    pltpu.sync_copy(sd_hbm.at[0, wid], sidx)
    pltpu.sync_copy(sd_hbm.at[1, wid], didx)
    plsc.subcore_barrier()

    for j in range(NDEP):
        pltpu.async_copy(r_hbm.at[sidx.at[j]], rows.at[j], sems[j])

    def quad(k, _):
        for j in range(NDEP):
            g = k * NDEP + j
            pltpu.make_async_copy(r_hbm.at[sidx.at[g]], rows.at[j], sems[j]).wait()
            pltpu.sync_copy(rows.at[j], acc.at[didx.at[g]], add=True)
            pltpu.async_copy(r_hbm.at[sidx.at[g + NDEP]], rows.at[j], sems[j])
        return 0

    lax.fori_loop(0, NBLK // NDEP - 1, quad, 0)
    for j in range(NDEP):
        g = NBLK - NDEP + j
        pltpu.make_async_copy(r_hbm.at[sidx.at[g]], rows.at[j], sems[j]).wait()
        pltpu.sync_copy(rows.at[j], acc.at[didx.at[g]], add=True)

    plsc.subcore_barrier()
    _flush_acc(acc, out_hbm, c, s)


@functools.partial(
    pl.kernel,
    out_type=jax.ShapeDtypeStruct((2, NPAD, HP), _f32),
    mesh=_mesh,
    compiler_params=_sc_params,
    scratch_types=[
        pltpu.VMEM((NBLK, BE), jnp.int32),
        pltpu.VMEM((NDEP, BE, HP), _f32),
        pltpu.VMEM((ZB, HP), _f32),
        pltpu.VMEM_SHARED((NPAD, HP), _f32),
    ] + [pltpu.SemaphoreType.DMA] * NDEP,
)
def _seg_linear(vals_hbm, sd_hbm, out_hbm, didx, rows, zbuf, acc, s0, s1, s2, s3):
    c = lax.axis_index("c")
    s = lax.axis_index("s")
    wid = s * 2 + c
    sems = (s0, s1, s2, s3)
    _zero_acc(zbuf, acc, s)
    pltpu.sync_copy(sd_hbm.at[1, wid], didx)
    plsc.subcore_barrier()

    def src_block(g):
        return vals_hbm.at[pl.ds((wid * NBLK + g) * BE, BE)]

    for j in range(NDEP):
        pltpu.async_copy(src_block(j), rows.at[j], sems[j])

    def quad(k, _):
        for j in range(NDEP):
            g = k * NDEP + j
            pltpu.make_async_copy(src_block(g), rows.at[j], sems[j]).wait()
            pltpu.sync_copy(rows.at[j], acc.at[didx.at[g]], add=True)
            pltpu.async_copy(src_block(g + NDEP), rows.at[j], sems[j])
        return 0

    lax.fori_loop(0, NBLK // NDEP - 1, quad, 0)
    for j in range(NDEP):
        g = NBLK - NDEP + j
        pltpu.make_async_copy(src_block(g), rows.at[j], sems[j]).wait()
        pltpu.sync_copy(rows.at[j], acc.at[didx.at[g]], add=True)

    plsc.subcore_barrier()
    _flush_acc(acc, out_hbm, c, s)


_EI_R = E // BE
_EI_RP = EPAD // BE


def _idx_body(e_ref, o_ref):
    i = pl.program_id(1)
    row = i * 256 + lax.broadcasted_iota(jnp.int32, (1, 256, 1), 1)
    o_ref[...] = jnp.where(row < _EI_R, e_ref[...], N)


_idx_call = pl.pallas_call(
    _idx_body,
    grid=(2, _EI_RP // 256),
    in_specs=[pl.BlockSpec((1, 256, BE), lambda p, i: (p, jnp.minimum(i, _EI_R // 256), 0))],
    out_specs=pl.BlockSpec((1, 256, BE), lambda p, i: (p, i, 0)),
    out_shape=jax.ShapeDtypeStruct((2, _EI_RP, BE), jnp.int32),
)

_EW_BLK = 512
_EW_GRID = 79


def _ew_body(ea_ref, w1_ref, w2_ref, e15_ref, o_ref):
    i = pl.program_id(0)
    t = jnp.maximum(jnp.dot(ea_ref[...], w1_ref[...], preferred_element_type=_f32), 0.0)
    m = jnp.maximum(jnp.dot(t, w2_ref[...], preferred_element_type=_f32), 0.0)
    row = i * _EW_BLK + lax.broadcasted_iota(jnp.int32, (_EW_BLK, 1), 0)
    o_ref[...] = jnp.where(row < ER8, m + e15_ref[...], 0.0)


_ew_call = pl.pallas_call(
    _ew_body,
    grid=(_EW_GRID,),
    in_specs=[
        pl.BlockSpec((_EW_BLK, 128), lambda i: (i, 0)),
        pl.BlockSpec((128, 128), lambda i: (0, 0)),
        pl.BlockSpec((128, 128), lambda i: (0, 0)),
        pl.BlockSpec((1, 128), lambda i: (0, 0)),
    ],
    out_specs=pl.BlockSpec((_EW_BLK, 128), lambda i: (i, 0)),
    out_shape=jax.ShapeDtypeStruct((EP8, 128), _f32),
)

_X_BLK = 256


def _x_body(x_ref, w_ref, b_ref, xp_ref, r0_ref):
    i = pl.program_id(0)
    xp = jnp.dot(x_ref[...], w_ref[...], preferred_element_type=_f32) + b_ref[...]
    row = i * _X_BLK + lax.broadcasted_iota(jnp.int32, (_X_BLK, 1), 0)
    xp = jnp.where(row < NR8, xp, 0.0)
    xp_ref[...] = xp
    r0_ref[...] = jnp.maximum(xp, 0.0)


_x_call = pl.pallas_call(
    _x_body,
    grid=(NP8 // _X_BLK,),
    in_specs=[
        pl.BlockSpec((_X_BLK, 8 * D_IN), lambda i: (i, 0)),
        pl.BlockSpec((8 * D_IN, 128), lambda i: (0, 0)),
        pl.BlockSpec((1, 128), lambda i: (0, 0)),
    ],
    out_specs=[
        pl.BlockSpec((_X_BLK, 128), lambda i: (i, 0)),
        pl.BlockSpec((_X_BLK, 128), lambda i: (i, 0)),
    ],
    out_shape=[
        jax.ShapeDtypeStruct((NP8, 128), _f32),
        jax.ShapeDtypeStruct((NP8, 128), _f32),
    ],
)

_C_BLK = 256
_C_GRID = NP8 // _C_BLK


def _combine_core(s0, s1, w0, w1, r, xp, av, bv, cv, uv, wxi, wh, psel):
    S = s0[...] + s1[...]
    W = w0[...] + w1[...]
    indeg = jnp.dot(W, psel[...], preferred_element_type=_f32)
    aggr = av[...] * r[...] * indeg + bv[...] * S + cv[...] * W
    h = aggr * uv[...]
    return (jnp.dot(xp[...], wxi[...], preferred_element_type=_f32)
            + jnp.dot(h, wh[...], preferred_element_type=_f32))


def _combine_body(s0, s1, w0, w1, r, xp, av, bv, cv, uv, wxi, wh, psel, out_r):
    out_r[...] = jnp.maximum(
        _combine_core(s0, s1, w0, w1, r, xp, av, bv, cv, uv, wxi, wh, psel), 0.0)


def _final_body(s0, s1, w0, w1, r, xp, av, bv, cv, uv, wxi, wh, psel, owx, owh, o_ref):
    h = _combine_core(s0, s1, w0, w1, r, xp, av, bv, cv, uv, wxi, wh, psel)
    rn = jnp.maximum(h, 0.0)
    o_ref[...] = (jnp.dot(xp[...], owx[...], preferred_element_type=_f32)
                  + jnp.dot(rn, owh[...], preferred_element_type=_f32))


def _nblk(i):
    return (i, 0)


def _nblk1(i):
    return (i + _C_GRID, 0)


def _small(i):
    return (0, 0)


_comb_in_specs = [
    pl.BlockSpec((_C_BLK, 128), _nblk),
    pl.BlockSpec((_C_BLK, 128), _nblk1),
    pl.BlockSpec((_C_BLK, 128), _nblk),
    pl.BlockSpec((_C_BLK, 128), _nblk1),
    pl.BlockSpec((_C_BLK, 128), _nblk),
    pl.BlockSpec((_C_BLK, 128), _nblk),
    pl.BlockSpec((1, 128), _small),
    pl.BlockSpec((1, 128), _small),
    pl.BlockSpec((1, 128), _small),
    pl.BlockSpec((1, 128), _small),
    pl.BlockSpec((128, 128), _small),
    pl.BlockSpec((128, 128), _small),
    pl.BlockSpec((128, 128), _small),
]

_combine_call = pl.pallas_call(
    _combine_body,
    grid=(_C_GRID,),
    in_specs=_comb_in_specs,
    out_specs=pl.BlockSpec((_C_BLK, 128), _nblk),
    out_shape=jax.ShapeDtypeStruct((NP8, 128), _f32),
)

_final_call = pl.pallas_call(
    _final_body,
    grid=(_C_GRID,),
    in_specs=_comb_in_specs + [
        pl.BlockSpec((128, 8), _small),
        pl.BlockSpec((128, 8), _small),
    ],
    out_specs=pl.BlockSpec((_C_BLK, 8), _nblk),
    out_shape=jax.ShapeDtypeStruct((NP8, 8), _f32),
)


def kernel(x, edge_index, edge_attr, ew_w1, ew_w2, i_w, i_b, conv_mlp, conv_upd, lin_ws, o_w):
    ei = edge_index.astype(jnp.int32).reshape(2, _EI_R, BE)
    sd3 = _idx_call(ei).reshape(2, NW, NBLK, BE)

    eye8 = jnp.eye(8, dtype=_f32)
    w1p = jnp.zeros((D_EDGE, HP), _f32).at[:, :H].set(ew_w1)
    w2p = jnp.zeros((HP, HP), _f32).at[:H, :H].set(ew_w2)
    w1b = jnp.kron(eye8, w1p)
    w2b = jnp.kron(eye8, w2p)
    e15t = jnp.tile(jnp.zeros((1, HP), _f32).at[0, HP - 1].set(1.0), (1, 8))

    iwp = jnp.zeros((D_IN, HP), _f32).at[:, :H].set(i_w)
    iw8 = jnp.kron(eye8, iwp)
    ib8 = jnp.tile(jnp.zeros((1, HP), _f32).at[0, :H].set(i_b), (1, 8))

    psel = jnp.kron(eye8, jnp.zeros((HP, HP), _f32).at[HP - 1, :].set(1.0))

    ea8 = edge_attr.reshape(ER8, 128)
    x8 = x.reshape(NR8, 8 * D_IN)

    ew = _ew_call(ea8, w1b, w2b, e15t)
    xp, r = _x_call(x8, iw8, ib8)

    sewp = _seg_linear(ew.reshape(EPAD, HP), sd3)
    sew8 = sewp.reshape(2 * NP8, 128)

    for l in range(L):
        Sp = _seg_gather(r.reshape(NPAD, HP), sd3)
        sp8 = Sp.reshape(2 * NP8, 128)
        av = jnp.tile(jnp.zeros((1, HP), _f32).at[0, :H].set(conv_mlp[l, :, 0]), (1, 8))
        bv = jnp.tile(jnp.zeros((1, HP), _f32).at[0, :H].set(conv_mlp[l, :, 1]), (1, 8))
        cv = jnp.tile(jnp.zeros((1, HP), _f32).at[0, :H].set(conv_mlp[l, :, 2]), (1, 8))
        uv = jnp.tile(jnp.zeros((1, HP), _f32).at[0, :H].set(conv_upd[l]), (1, 8))
        wx = jnp.zeros((HP, HP), _f32).at[:H, :H].set(lin_ws[l, :H, :])
        wh = jnp.zeros((HP, HP), _f32).at[:H, :H].set(lin_ws[l, H:, :])
        wxi = jnp.kron(eye8, wx) + jnp.eye(128, dtype=_f32)
        whb = jnp.kron(eye8, wh)
        args = (sp8, sp8, sew8, sew8, r, xp, av, bv, cv, uv, wxi, whb, psel)
        if l < L - 1:
            r = _combine_call(*args)
        else:
            owx = jnp.kron(eye8, jnp.zeros((HP, 1), _f32).at[:H, :].set(o_w[:H, :]))
            owh = jnp.kron(eye8, jnp.zeros((HP, 1), _f32).at[:H, :].set(o_w[H:, :]))
            out8 = _final_call(*args, owx, owh)
    return out8.reshape(NPAD, 1)[:N]


# speedup vs baseline: 1.0525x; 1.0525x over previous
import functools

import jax
import jax.numpy as jnp
from jax import lax
from jax.experimental import pallas as pl
from jax.experimental.pallas import tpu as pltpu
from jax.experimental.pallas import tpu_sc as plsc

N = 10000
E = 320000
D_IN = 128
D_EDGE = 16
H = 10
L = 3

HP = 16
NW = 32
BE = 128
NBLK = 80
EPAD = NW * NBLK * BE
NPAD = 10240
RPS = NPAD // 16
ZB = 64
NDEP = 4

ER8 = E // 8
EP8 = EPAD // 8
NR8 = N // 8
NP8 = NPAD // 8

_f32 = jnp.float32


_mesh = plsc.VectorSubcoreMesh(core_axis_name="c", subcore_axis_name="s")
_sc_params = pltpu.CompilerParams(use_tc_tiling_on_sc=False)


def _zero_acc(zbuf, acc, s):
    def zrow(i, _):
        zbuf[i] = jnp.zeros((HP,), _f32)
        return 0

    lax.fori_loop(0, ZB, zrow, 0)

    def zcp(k, _):
        pltpu.sync_copy(zbuf, acc.at[pl.ds(s * RPS + k * ZB, ZB)])
        return 0

    lax.fori_loop(0, RPS // ZB, zcp, 0)


def _flush_acc(acc, out_hbm, c, s):
    pltpu.sync_copy(
        acc.at[pl.ds(s * RPS, RPS)],
        out_hbm.at[c, pl.ds(s * RPS, RPS)],
    )


@functools.partial(
    pl.kernel,
    out_type=jax.ShapeDtypeStruct((2, NPAD, HP), _f32),
    mesh=_mesh,
    compiler_params=_sc_params,
    scratch_types=[
        pltpu.VMEM((NBLK, BE), jnp.int32),
        pltpu.VMEM((NBLK, BE), jnp.int32),
        pltpu.VMEM((NDEP, BE, HP), _f32),
        pltpu.VMEM((ZB, HP), _f32),
        pltpu.VMEM_SHARED((NPAD, HP), _f32),
    ] + [pltpu.SemaphoreType.DMA] * (NDEP + 2),
)
def _seg_gather(r_hbm, sd_hbm, out_hbm, sidx, didx, rows, zbuf, acc,
                g0, g1, g2, g3, sc0, sc1):
    c = lax.axis_index("c")
    s = lax.axis_index("s")
    wid = s * 2 + c
    gsem = (g0, g1, g2, g3)
    ssem = (sc0, sc1)
    _zero_acc(zbuf, acc, s)
    pltpu.sync_copy(sd_hbm.at[0, wid], sidx)
    pltpu.sync_copy(sd_hbm.at[1, wid], didx)
    plsc.subcore_barrier()

    def gath(g, j):
        pltpu.async_copy(r_hbm.at[sidx.at[g]], rows.at[j], gsem[j])

    def wait_gath(g, j):
        pltpu.make_async_copy(r_hbm.at[sidx.at[g]], rows.at[j], gsem[j]).wait()

    def scat(g, j, p):
        pltpu.async_copy(rows.at[j], acc.at[didx.at[g]], ssem[p], add=True)

    def wait_scat(g, j, p):
        pltpu.make_async_copy(rows.at[j], acc.at[didx.at[g]], ssem[p]).wait()

    gath(0, 0)
    gath(1, 1)
    for j in range(NDEP):
        g = j
        wait_gath(g, j)
        if j >= 2:
            wait_scat(g - 2, j - 2, j % 2)
        scat(g, j, j % 2)
        gath(g + 2, (g + 2) % NDEP)

    def quad(k, _):
        for j in range(NDEP):
            g = k * NDEP + j
            wait_gath(g, j)
            wait_scat(g - 2, (j + 2) % NDEP, j % 2)
            scat(g, j, j % 2)
            gath(g + 2, (j + 2) % NDEP)
        return 0

    lax.fori_loop(1, NBLK // NDEP - 1, quad, 0)
    for j in range(NDEP):
        g = NBLK - NDEP + j
        wait_gath(g, j)
        wait_scat(g - 2, (j + 2) % NDEP, j % 2)
        scat(g, j, j % 2)
        if j < 2:
            gath(g + 2, (g + 2) % NDEP)
    wait_scat(NBLK - 2, 2, 0)
    wait_scat(NBLK - 1, 3, 1)

    plsc.subcore_barrier()
    _flush_acc(acc, out_hbm, c, s)


@functools.partial(
    pl.kernel,
    out_type=jax.ShapeDtypeStruct((2, NPAD, HP), _f32),
    mesh=_mesh,
    compiler_params=_sc_params,
    scratch_types=[
        pltpu.VMEM((NBLK, BE), jnp.int32),
        pltpu.VMEM((NDEP, BE, HP), _f32),
        pltpu.VMEM((ZB, HP), _f32),
        pltpu.VMEM_SHARED((NPAD, HP), _f32),
    ] + [pltpu.SemaphoreType.DMA] * (NDEP + 2),
)
def _seg_linear(vals_hbm, sd_hbm, out_hbm, didx, rows, zbuf, acc,
                g0, g1, g2, g3, sc0, sc1):
    c = lax.axis_index("c")
    s = lax.axis_index("s")
    wid = s * 2 + c
    gsem = (g0, g1, g2, g3)
    ssem = (sc0, sc1)
    _zero_acc(zbuf, acc, s)
    pltpu.sync_copy(sd_hbm.at[1, wid], didx)
    plsc.subcore_barrier()

    def src_block(g):
        return vals_hbm.at[pl.ds((wid * NBLK + g) * BE, BE)]

    def gath(g, j):
        pltpu.async_copy(src_block(g), rows.at[j], gsem[j])

    def wait_gath(g, j):
        pltpu.make_async_copy(src_block(g), rows.at[j], gsem[j]).wait()

    def scat(g, j, p):
        pltpu.async_copy(rows.at[j], acc.at[didx.at[g]], ssem[p], add=True)

    def wait_scat(g, j, p):
        pltpu.make_async_copy(rows.at[j], acc.at[didx.at[g]], ssem[p]).wait()

    gath(0, 0)
    gath(1, 1)
    for j in range(NDEP):
        g = j
        wait_gath(g, j)
        if j >= 2:
            wait_scat(g - 2, j - 2, j % 2)
        scat(g, j, j % 2)
        gath(g + 2, (g + 2) % NDEP)

    def quad(k, _):
        for j in range(NDEP):
            g = k * NDEP + j
            wait_gath(g, j)
            wait_scat(g - 2, (j + 2) % NDEP, j % 2)
            scat(g, j, j % 2)
            gath(g + 2, (j + 2) % NDEP)
        return 0

    lax.fori_loop(1, NBLK // NDEP - 1, quad, 0)
    for j in range(NDEP):
        g = NBLK - NDEP + j
        wait_gath(g, j)
        wait_scat(g - 2, (j + 2) % NDEP, j % 2)
        scat(g, j, j % 2)
        if j < 2:
            gath(g + 2, (g + 2) % NDEP)
    wait_scat(NBLK - 2, 2, 0)
    wait_scat(NBLK - 1, 3, 1)

    plsc.subcore_barrier()
    _flush_acc(acc, out_hbm, c, s)


_EI_R = E // BE
_EI_RP = EPAD // BE


def _idx_body(e_ref, o_ref):
    i = pl.program_id(1)
    row = i * 256 + lax.broadcasted_iota(jnp.int32, (1, 256, 1), 1)
    o_ref[...] = jnp.where(row < _EI_R, e_ref[...], N)


_idx_call = pl.pallas_call(
    _idx_body,
    grid=(2, _EI_RP // 256),
    in_specs=[pl.BlockSpec((1, 256, BE), lambda p, i: (p, jnp.minimum(i, _EI_R // 256), 0))],
    out_specs=pl.BlockSpec((1, 256, BE), lambda p, i: (p, i, 0)),
    out_shape=jax.ShapeDtypeStruct((2, _EI_RP, BE), jnp.int32),
)

_EW_BLK = 2048
_EW_GRID = EP8 // 2048


def _ew_body(ea_ref, w1_ref, w2_ref, e15_ref, o_ref):
    i = pl.program_id(0)
    t = jnp.maximum(jnp.dot(ea_ref[...], w1_ref[...], preferred_element_type=_f32), 0.0)
    m = jnp.maximum(jnp.dot(t, w2_ref[...], preferred_element_type=_f32), 0.0)
    row = i * _EW_BLK + lax.broadcasted_iota(jnp.int32, (_EW_BLK, 1), 0)
    o_ref[...] = jnp.where(row < ER8, m + e15_ref[...], 0.0)


_ew_call = pl.pallas_call(
    _ew_body,
    grid=(_EW_GRID,),
    in_specs=[
        pl.BlockSpec((_EW_BLK, 128), lambda i: (i, 0)),
        pl.BlockSpec((128, 128), lambda i: (0, 0)),
        pl.BlockSpec((128, 128), lambda i: (0, 0)),
        pl.BlockSpec((1, 128), lambda i: (0, 0)),
    ],
    out_specs=pl.BlockSpec((_EW_BLK, 128), lambda i: (i, 0)),
    out_shape=jax.ShapeDtypeStruct((EP8, 128), _f32),
)

_X_BLK = 256


def _x_body(x_ref, w_ref, b_ref, xp_ref, r0_ref):
    i = pl.program_id(0)
    xp = jnp.dot(x_ref[...], w_ref[...], preferred_element_type=_f32) + b_ref[...]
    row = i * _X_BLK + lax.broadcasted_iota(jnp.int32, (_X_BLK, 1), 0)
    xp = jnp.where(row < NR8, xp, 0.0)
    xp_ref[...] = xp
    r0_ref[...] = jnp.maximum(xp, 0.0)


_x_call = pl.pallas_call(
    _x_body,
    grid=(NP8 // _X_BLK,),
    in_specs=[
        pl.BlockSpec((_X_BLK, 8 * D_IN), lambda i: (i, 0)),
        pl.BlockSpec((8 * D_IN, 128), lambda i: (0, 0)),
        pl.BlockSpec((1, 128), lambda i: (0, 0)),
    ],
    out_specs=[
        pl.BlockSpec((_X_BLK, 128), lambda i: (i, 0)),
        pl.BlockSpec((_X_BLK, 128), lambda i: (i, 0)),
    ],
    out_shape=[
        jax.ShapeDtypeStruct((NP8, 128), _f32),
        jax.ShapeDtypeStruct((NP8, 128), _f32),
    ],
)

_C_BLK = 256
_C_GRID = NP8 // _C_BLK


def _combine_core(s0, s1, w0, w1, r, xp, av, bv, cv, uv, wxi, wh, psel):
    S = s0[...] + s1[...]
    W = w0[...] + w1[...]
    indeg = jnp.dot(W, psel[...], preferred_element_type=_f32)
    aggr = av[...] * r[...] * indeg + bv[...] * S + cv[...] * W
    h = aggr * uv[...]
    return (jnp.dot(xp[...], wxi[...], preferred_element_type=_f32)
            + jnp.dot(h, wh[...], preferred_element_type=_f32))


def _combine_body(s0, s1, w0, w1, r, xp, av, bv, cv, uv, wxi, wh, psel, out_r):
    out_r[...] = jnp.maximum(
        _combine_core(s0, s1, w0, w1, r, xp, av, bv, cv, uv, wxi, wh, psel), 0.0)


def _final_body(s0, s1, w0, w1, r, xp, av, bv, cv, uv, wxi, wh, psel, owx, owh, o_ref):
    h = _combine_core(s0, s1, w0, w1, r, xp, av, bv, cv, uv, wxi, wh, psel)
    rn = jnp.maximum(h, 0.0)
    o_ref[...] = (jnp.dot(xp[...], owx[...], preferred_element_type=_f32)
                  + jnp.dot(rn, owh[...], preferred_element_type=_f32))


def _nblk(i):
    return (i, 0)


def _nblk1(i):
    return (i + _C_GRID, 0)


def _small(i):
    return (0, 0)


_comb_in_specs = [
    pl.BlockSpec((_C_BLK, 128), _nblk),
    pl.BlockSpec((_C_BLK, 128), _nblk1),
    pl.BlockSpec((_C_BLK, 128), _nblk),
    pl.BlockSpec((_C_BLK, 128), _nblk1),
    pl.BlockSpec((_C_BLK, 128), _nblk),
    pl.BlockSpec((_C_BLK, 128), _nblk),
    pl.BlockSpec((1, 128), _small),
    pl.BlockSpec((1, 128), _small),
    pl.BlockSpec((1, 128), _small),
    pl.BlockSpec((1, 128), _small),
    pl.BlockSpec((128, 128), _small),
    pl.BlockSpec((128, 128), _small),
    pl.BlockSpec((128, 128), _small),
]

_combine_call = pl.pallas_call(
    _combine_body,
    grid=(_C_GRID,),
    in_specs=_comb_in_specs,
    out_specs=pl.BlockSpec((_C_BLK, 128), _nblk),
    out_shape=jax.ShapeDtypeStruct((NP8, 128), _f32),
)

_final_call = pl.pallas_call(
    _final_body,
    grid=(_C_GRID,),
    in_specs=_comb_in_specs + [
        pl.BlockSpec((128, 8), _small),
        pl.BlockSpec((128, 8), _small),
    ],
    out_specs=pl.BlockSpec((_C_BLK, 8), _nblk),
    out_shape=jax.ShapeDtypeStruct((NP8, 8), _f32),
)


def kernel(x, edge_index, edge_attr, ew_w1, ew_w2, i_w, i_b, conv_mlp, conv_upd, lin_ws, o_w):
    ei = edge_index.astype(jnp.int32).reshape(2, _EI_R, BE)
    sd3 = _idx_call(ei).reshape(2, NW, NBLK, BE)

    eye8 = jnp.eye(8, dtype=_f32)
    w1p = jnp.zeros((D_EDGE, HP), _f32).at[:, :H].set(ew_w1)
    w2p = jnp.zeros((HP, HP), _f32).at[:H, :H].set(ew_w2)
    w1b = jnp.kron(eye8, w1p)
    w2b = jnp.kron(eye8, w2p)
    e15t = jnp.tile(jnp.zeros((1, HP), _f32).at[0, HP - 1].set(1.0), (1, 8))

    iwp = jnp.zeros((D_IN, HP), _f32).at[:, :H].set(i_w)
    iw8 = jnp.kron(eye8, iwp)
    ib8 = jnp.tile(jnp.zeros((1, HP), _f32).at[0, :H].set(i_b), (1, 8))

    psel = jnp.kron(eye8, jnp.zeros((HP, HP), _f32).at[HP - 1, :].set(1.0))

    ea8 = edge_attr.reshape(ER8, 128)
    x8 = x.reshape(NR8, 8 * D_IN)

    ew = _ew_call(ea8, w1b, w2b, e15t)
    xp, r = _x_call(x8, iw8, ib8)

    sewp = _seg_linear(ew.reshape(EPAD, HP), sd3)
    sew8 = sewp.reshape(2 * NP8, 128)

    for l in range(L):
        Sp = _seg_gather(r.reshape(NPAD, HP), sd3)
        sp8 = Sp.reshape(2 * NP8, 128)
        av = jnp.tile(jnp.zeros((1, HP), _f32).at[0, :H].set(conv_mlp[l, :, 0]), (1, 8))
        bv = jnp.tile(jnp.zeros((1, HP), _f32).at[0, :H].set(conv_mlp[l, :, 1]), (1, 8))
        cv = jnp.tile(jnp.zeros((1, HP), _f32).at[0, :H].set(conv_mlp[l, :, 2]), (1, 8))
        uv = jnp.tile(jnp.zeros((1, HP), _f32).at[0, :H].set(conv_upd[l]), (1, 8))
        wx = jnp.zeros((HP, HP), _f32).at[:H, :H].set(lin_ws[l, :H, :])
        wh = jnp.zeros((HP, HP), _f32).at[:H, :H].set(lin_ws[l, H:, :])
        wxi = jnp.kron(eye8, wx) + jnp.eye(128, dtype=_f32)
        whb = jnp.kron(eye8, wh)
        args = (sp8, sp8, sew8, sew8, r, xp, av, bv, cv, uv, wxi, whb, psel)
        if l < L - 1:
            r = _combine_call(*args)
        else:
            owx = jnp.kron(eye8, jnp.zeros((HP, 1), _f32).at[:H, :].set(o_w[:H, :]))
            owh = jnp.kron(eye8, jnp.zeros((HP, 1), _f32).at[:H, :].set(o_w[H:, :]))
            out8 = _final_call(*args, owx, owh)
    return out8.reshape(NPAD, 1)[:N]
